# baseline (speedup 1.0000x reference)
import functools

import jax
import jax.numpy as jnp
from jax import lax
from jax.experimental import pallas as pl
from jax.experimental.pallas import tpu as pltpu
from jax.experimental.pallas import tpu_sc as plsc

NC = 2
NS = 16
NT = NC * NS
LANE = 16
C = 128


def _mesh():
    return plsc.VectorSubcoreMesh(core_axis_name="c", subcore_axis_name="s")


def _degrees(edges_r, n_pad, e_pad):
    nb = e_pad // C // NS
    nps = n_pad // NS

    @functools.partial(
        pl.kernel,
        out_type=jax.ShapeDtypeStruct((NC, n_pad), jnp.float32),
        mesh=_mesh(),
        scratch_types=[
            pltpu.VMEM((nb, C), jnp.int32),
            pltpu.VMEM((C,), jnp.float32),
            pltpu.VMEM((nps,), jnp.float32),
            pltpu.VMEM_SHARED((n_pad,), jnp.float32),
            pltpu.SemaphoreType.DMA,
        ],
    )
    def deg_kernel(edges_hbm, out_hbm, idx_v, ones_v, zero_v, cnt_sh, sem):
        c = lax.axis_index("c")
        s = lax.axis_index("s")
        cp = pltpu.async_copy(edges_hbm.at[c, pl.ds(s * nb, nb)], idx_v, sem)

        @pl.loop(0, C, step=LANE)
        def _(j):
            ones_v[pl.ds(j, LANE)] = jnp.ones((LANE,), jnp.float32)

        @pl.loop(0, nps, step=LANE)
        def _(j):
            zero_v[pl.ds(j, LANE)] = jnp.zeros((LANE,), jnp.float32)

        pltpu.sync_copy(zero_v, cnt_sh.at[pl.ds(s * nps, nps)])
        cp.wait()
        plsc.subcore_barrier()

        @pl.loop(0, nb)
        def _(j):
            pltpu.sync_copy(ones_v, cnt_sh.at[idx_v.at[j]], add=True)

        plsc.subcore_barrier()
        pltpu.sync_copy(cnt_sh.at[pl.ds(s * nps, nps)],
                        out_hbm.at[c, pl.ds(s * nps, nps)])

    return deg_kernel(edges_r)


def _aggregate(h_s, edges_r, n_pad, e_pad, d):
    edges_r = edges_r.reshape(2, -1)
    cb = 80
    nd = 4
    nch = e_pad // cb // NT
    nq = 8
    nchq = nch // nq
    nps = n_pad // NS

    @functools.partial(
        pl.kernel,
        out_type=[jax.ShapeDtypeStruct((n_pad, d), jnp.float32),
                  jax.ShapeDtypeStruct((n_pad, d), jnp.float32)],
        mesh=_mesh(),
        scratch_types=[
            pltpu.VMEM((nchq, cb), jnp.int32),
            pltpu.VMEM((nchq, cb), jnp.int32),
            [pltpu.VMEM((cb, d), jnp.float32) for _ in range(nd)],
            pltpu.VMEM_SHARED((n_pad, d), jnp.float32),
            [pltpu.SemaphoreType.DMA for _ in range(nd)],
            [pltpu.SemaphoreType.DMA for _ in range(nd)],
        ],
    )
    def agg_kernel(h_hbm, edges_hbm, out_lo, out_hi,
                   sidx, didx, rows, acc_sh, gs, ss):
        c = lax.axis_index("c")
        s = lax.axis_index("s")
        base = (c * NS + s) * nch
        sl = pl.ds(s * nps, nps)

        cp0 = pltpu.async_copy(edges_hbm.at[0, pl.ds(base, nchq)], sidx,
                               gs[0])
        cp1 = pltpu.async_copy(edges_hbm.at[1, pl.ds(base, nchq)], didx,
                               gs[1])

        @pl.loop(0, cb)
        def _(i):
            @pl.loop(0, d, step=LANE)
            def _(j):
                rows[0][i, pl.ds(j, LANE)] = jnp.zeros((LANE,), jnp.float32)

        nzc = nps // cb
        for z in range(nzc):
            pltpu.async_copy(rows[0], acc_sh.at[pl.ds(s * nps + z * cb, cb)],
                             ss[z % nd])
        for z in range(nzc):
            pltpu.make_async_copy(rows[0],
                                  acc_sh.at[pl.ds(s * nps + z * cb, cb)],
                                  ss[z % nd]).wait()
        cp0.wait()
        cp1.wait()

        plsc.subcore_barrier()

        for q in range(nq):
            if q:
                qb = base + q * nchq
                pltpu.sync_copy(edges_hbm.at[0, pl.ds(qb, nchq)], sidx)
                pltpu.sync_copy(edges_hbm.at[1, pl.ds(qb, nchq)], didx)

            for k in range(nd):
                pltpu.async_copy(h_hbm.at[sidx.at[k]], rows[k], gs[k])

            @pl.loop(0, nchq - nd, step=nd)
            def _(j):
                for k in range(nd):
                    pltpu.make_async_copy(h_hbm.at[sidx.at[j + k]],
                                          rows[k], gs[k]).wait()
                    pltpu.async_copy(rows[k], acc_sh.at[didx.at[j + k]],
                                     ss[k], add=True)
                for k in range(nd):
                    pltpu.make_async_copy(rows[k],
                                          acc_sh.at[didx.at[j + k]],
                                          ss[k]).wait()
                    pltpu.async_copy(h_hbm.at[sidx.at[j + nd + k]],
                                     rows[k], gs[k])

            for k in range(nd):
                jj = nchq - nd + k
                pltpu.make_async_copy(h_hbm.at[sidx.at[jj]], rows[k],
                                      gs[k]).wait()
                pltpu.sync_copy(rows[k], acc_sh.at[didx.at[jj]], add=True)

        plsc.subcore_barrier()

        @pl.when(c == 0)
        def _():
            pltpu.sync_copy(acc_sh.at[sl], out_lo.at[sl])

        @pl.when(c != 0)
        def _():
            pltpu.sync_copy(acc_sh.at[sl], out_hi.at[sl])

        @pl.when(c == 0)
        def _():
            pltpu.sync_copy(acc_sh.at[sl], out_lo.at[sl])

        @pl.when(c != 0)
        def _():
            pltpu.sync_copy(acc_sh.at[sl], out_hi.at[sl])

    return agg_kernel(h_s, edges_r.reshape(2, e_pad // cb, cb))


def _prep(x_pad, ocnt, n, n_pad, d):
    r = 512

    def body(x_ref, oc_ref, hs_ref):
        i = pl.program_id(0)
        rows = i * r + lax.broadcasted_iota(jnp.int32, (r, 1), 0)
        onorm = jnp.where(rows < n,
                          lax.rsqrt(jnp.maximum(oc_ref[...], 1.0)), 0.0)
        hs_ref[...] = x_ref[...] * onorm

    return pl.pallas_call(
        body,
        grid=(n_pad // r,),
        in_specs=[pl.BlockSpec((r, d), lambda i: (i, 0)),
                  pl.BlockSpec((r, 1), lambda i: (i, 0))],
        out_specs=pl.BlockSpec((r, d), lambda i: (i, 0)),
        out_shape=jax.ShapeDtypeStruct((n_pad, d), jnp.float32),
    )(x_pad, ocnt)


def _dense(p_lo, p_hi, icnt, ocnt, w, b, n, n_pad, d):
    r = 512

    def body(a0_ref, a1_ref, ic_ref, oc_ref, w_ref, b_ref,
             y_ref, ys_ref, sum_ref):
        i = pl.program_id(0)
        rows = i * r + lax.broadcasted_iota(jnp.int32, (r, 1), 0)
        innorm = lax.rsqrt(jnp.maximum(ic_ref[...], 1.0))
        onorm = jnp.where(rows < n,
                          lax.rsqrt(jnp.maximum(oc_ref[...], 1.0)), 0.0)
        a = (a0_ref[...] + a1_ref[...]) * innorm
        yv = jnp.dot(a, w_ref[...], preferred_element_type=jnp.float32)
        yv = jnp.maximum(yv + b_ref[...], 0.0)
        y_ref[...] = yv
        ys_ref[...] = yv * onorm
        sum_ref[...] = jnp.sum(yv, axis=1, keepdims=True)

    return pl.pallas_call(
        body,
        grid=(n_pad // r,),
        in_specs=[pl.BlockSpec((r, d), lambda i: (i, 0)),
                  pl.BlockSpec((r, d), lambda i: (i, 0)),
                  pl.BlockSpec((r, 1), lambda i: (i, 0)),
                  pl.BlockSpec((r, 1), lambda i: (i, 0)),
                  pl.BlockSpec((d, d), lambda i: (0, 0)),
                  pl.BlockSpec((1, d), lambda i: (0, 0))],
        out_specs=[pl.BlockSpec((r, d), lambda i: (i, 0)),
                   pl.BlockSpec((r, d), lambda i: (i, 0)),
                   pl.BlockSpec((r, 1), lambda i: (i, 0))],
        out_shape=[jax.ShapeDtypeStruct((n_pad, d), jnp.float32),
                   jax.ShapeDtypeStruct((n_pad, d), jnp.float32),
                   jax.ShapeDtypeStruct((n_pad, 1), jnp.float32)],
    )(p_lo, p_hi, icnt, ocnt, w, b)


def kernel(x, edge_index, graph_len, W1, b1, W2, b2, W3, b3):
    n, d = x.shape
    e = edge_index.shape[1]
    n_pad = -(-(n + 1) // 512) * 512
    e_pad = -(-e // (NT * C * 2)) * (NT * C * 2)

    pad_idx = n + jnp.arange(e_pad - e, dtype=jnp.int32) % (n_pad - n)
    edges = jnp.concatenate(
        [edge_index, jnp.stack([pad_idx, pad_idx])], axis=1)
    edges_r = edges.reshape(2, e_pad // C, C)
    x_pad = jnp.pad(x, ((0, n_pad - n), (0, 0)))

    cnt = _degrees(edges_r, n_pad, e_pad)
    ocnt = cnt[0].reshape(n_pad, 1)
    icnt = cnt[1].reshape(n_pad, 1)

    h_s = _prep(x_pad, ocnt, n, n_pad, d)
    ys, sums = [], []
    for (w, b) in ((W1, b1), (W2, b2), (W3, b3)):
        p_lo, p_hi = _aggregate(h_s, edges_r, n_pad, e_pad, d)
        y, h_s, ysum = _dense(p_lo, p_hi, icnt, ocnt, w,
                              b.reshape(1, d), n, n_pad, d)
        ys.append(y)
        sums.append(ysum)

    x_out = jnp.concatenate([t[:n, 0] for t in sums], axis=0)
    xs_cat = jnp.concatenate([t[:n] for t in ys], axis=1)
    return (x_out, xs_cat)


# speedup vs baseline: 1.0899x; 1.0899x over previous
import functools

import jax
import jax.numpy as jnp
from jax import lax
from jax.experimental import pallas as pl
from jax.experimental.pallas import tpu as pltpu
from jax.experimental.pallas import tpu_sc as plsc

NC = 2
NS = 16
NT = NC * NS
LANE = 16
C = 128


def _mesh():
    return plsc.VectorSubcoreMesh(core_axis_name="c", subcore_axis_name="s")


def _degrees(edges_r, n_pad, e_pad):
    nb = e_pad // C // NS
    nps = n_pad // NS

    @functools.partial(
        pl.kernel,
        out_type=jax.ShapeDtypeStruct((NC, n_pad), jnp.float32),
        mesh=_mesh(),
        scratch_types=[
            pltpu.VMEM((nb, C), jnp.int32),
            pltpu.VMEM((C,), jnp.float32),
            pltpu.VMEM((nps,), jnp.float32),
            pltpu.VMEM_SHARED((n_pad,), jnp.float32),
            pltpu.SemaphoreType.DMA,
        ],
    )
    def deg_kernel(edges_hbm, out_hbm, idx_v, ones_v, zero_v, cnt_sh, sem):
        c = lax.axis_index("c")
        s = lax.axis_index("s")
        cp = pltpu.async_copy(edges_hbm.at[c, pl.ds(s * nb, nb)], idx_v, sem)

        @pl.loop(0, C, step=LANE)
        def _(j):
            ones_v[pl.ds(j, LANE)] = jnp.ones((LANE,), jnp.float32)

        @pl.loop(0, nps, step=LANE)
        def _(j):
            zero_v[pl.ds(j, LANE)] = jnp.zeros((LANE,), jnp.float32)

        pltpu.sync_copy(zero_v, cnt_sh.at[pl.ds(s * nps, nps)])
        cp.wait()
        plsc.subcore_barrier()

        @pl.loop(0, nb)
        def _(j):
            pltpu.sync_copy(ones_v, cnt_sh.at[idx_v.at[j]], add=True)

        plsc.subcore_barrier()
        pltpu.sync_copy(cnt_sh.at[pl.ds(s * nps, nps)],
                        out_hbm.at[c, pl.ds(s * nps, nps)])

    return deg_kernel(edges_r)


def _aggregate(h_s, edges_r, n_pad, e_pad, d):
    edges_r = edges_r.reshape(2, -1)
    cb = 64
    nd = 4
    nch = e_pad // cb // NT
    nq = 4
    nchq = nch // nq
    nps = n_pad // NS

    @functools.partial(
        pl.kernel,
        out_type=[jax.ShapeDtypeStruct((n_pad, d), jnp.float32),
                  jax.ShapeDtypeStruct((n_pad, d), jnp.float32)],
        mesh=_mesh(),
        scratch_types=[
            pltpu.VMEM((nchq, cb), jnp.int32),
            pltpu.VMEM((nchq, cb), jnp.int32),
            [pltpu.VMEM((cb, d), jnp.float32) for _ in range(nd)],
            pltpu.VMEM_SHARED((n_pad, d), jnp.float32),
            [pltpu.SemaphoreType.DMA for _ in range(nd)],
            [pltpu.SemaphoreType.DMA for _ in range(nd)],
        ],
    )
    def agg_kernel(h_hbm, edges_hbm, out_lo, out_hi,
                   sidx, didx, rows, acc_sh, gs, ss):
        c = lax.axis_index("c")
        s = lax.axis_index("s")
        base = (c * NS + s) * nch
        sl = pl.ds(s * nps, nps)

        cp0 = pltpu.async_copy(edges_hbm.at[0, pl.ds(base, nchq)], sidx,
                               gs[0])
        cp1 = pltpu.async_copy(edges_hbm.at[1, pl.ds(base, nchq)], didx,
                               gs[1])

        @pl.loop(0, cb)
        def _(i):
            @pl.loop(0, d, step=LANE)
            def _(j):
                rows[0][i, pl.ds(j, LANE)] = jnp.zeros((LANE,), jnp.float32)

        nzc = nps // cb
        for z in range(nzc):
            pltpu.async_copy(rows[0], acc_sh.at[pl.ds(s * nps + z * cb, cb)],
                             ss[z % nd])
        for z in range(nzc):
            pltpu.make_async_copy(rows[0],
                                  acc_sh.at[pl.ds(s * nps + z * cb, cb)],
                                  ss[z % nd]).wait()
        cp0.wait()
        cp1.wait()

        plsc.subcore_barrier()

        for q in range(nq):
            if q:
                qb = base + q * nchq
                pltpu.sync_copy(edges_hbm.at[0, pl.ds(qb, nchq)], sidx)
                pltpu.sync_copy(edges_hbm.at[1, pl.ds(qb, nchq)], didx)

            for k in range(nd):
                pltpu.async_copy(h_hbm.at[sidx.at[k]], rows[k], gs[k])

            @pl.loop(0, nchq - nd, step=nd)
            def _(j):
                for k in range(nd):
                    pltpu.make_async_copy(h_hbm.at[sidx.at[j + k]],
                                          rows[k], gs[k]).wait()
                    pltpu.async_copy(rows[k], acc_sh.at[didx.at[j + k]],
                                     ss[k], add=True)
                for k in range(nd):
                    pltpu.make_async_copy(rows[k],
                                          acc_sh.at[didx.at[j + k]],
                                          ss[k]).wait()
                    pltpu.async_copy(h_hbm.at[sidx.at[j + nd + k]],
                                     rows[k], gs[k])

            for k in range(nd):
                jj = nchq - nd + k
                pltpu.make_async_copy(h_hbm.at[sidx.at[jj]], rows[k],
                                      gs[k]).wait()
                pltpu.sync_copy(rows[k], acc_sh.at[didx.at[jj]], add=True)

        plsc.subcore_barrier()

        @pl.when(c == 0)
        def _():
            pltpu.sync_copy(acc_sh.at[sl], out_lo.at[sl])

        @pl.when(c != 0)
        def _():
            pltpu.sync_copy(acc_sh.at[sl], out_hi.at[sl])

    return agg_kernel(h_s, edges_r.reshape(2, e_pad // cb, cb))


def _prep(x_pad, ocnt, n, n_pad, d):
    r = 512

    def body(x_ref, oc_ref, hs_ref):
        i = pl.program_id(0)
        rows = i * r + lax.broadcasted_iota(jnp.int32, (r, 1), 0)
        onorm = jnp.where(rows < n,
                          lax.rsqrt(jnp.maximum(oc_ref[...], 1.0)), 0.0)
        hs_ref[...] = x_ref[...] * onorm

    return pl.pallas_call(
        body,
        grid=(n_pad // r,),
        in_specs=[pl.BlockSpec((r, d), lambda i: (i, 0)),
                  pl.BlockSpec((r, 1), lambda i: (i, 0))],
        out_specs=pl.BlockSpec((r, d), lambda i: (i, 0)),
        out_shape=jax.ShapeDtypeStruct((n_pad, d), jnp.float32),
    )(x_pad, ocnt)


def _dense(p_lo, p_hi, icnt, ocnt, w, b, n, n_pad, d):
    r = 512

    def body(a0_ref, a1_ref, ic_ref, oc_ref, w_ref, b_ref,
             y_ref, ys_ref, sum_ref):
        i = pl.program_id(0)
        rows = i * r + lax.broadcasted_iota(jnp.int32, (r, 1), 0)
        innorm = lax.rsqrt(jnp.maximum(ic_ref[...], 1.0))
        onorm = jnp.where(rows < n,
                          lax.rsqrt(jnp.maximum(oc_ref[...], 1.0)), 0.0)
        a = (a0_ref[...] + a1_ref[...]) * innorm
        yv = jnp.dot(a, w_ref[...], preferred_element_type=jnp.float32)
        yv = jnp.maximum(yv + b_ref[...], 0.0)
        y_ref[...] = yv
        ys_ref[...] = yv * onorm
        sum_ref[...] = jnp.sum(yv, axis=1, keepdims=True)

    return pl.pallas_call(
        body,
        grid=(n_pad // r,),
        in_specs=[pl.BlockSpec((r, d), lambda i: (i, 0)),
                  pl.BlockSpec((r, d), lambda i: (i, 0)),
                  pl.BlockSpec((r, 1), lambda i: (i, 0)),
                  pl.BlockSpec((r, 1), lambda i: (i, 0)),
                  pl.BlockSpec((d, d), lambda i: (0, 0)),
                  pl.BlockSpec((1, d), lambda i: (0, 0))],
        out_specs=[pl.BlockSpec((r, d), lambda i: (i, 0)),
                   pl.BlockSpec((r, d), lambda i: (i, 0)),
                   pl.BlockSpec((r, 1), lambda i: (i, 0))],
        out_shape=[jax.ShapeDtypeStruct((n_pad, d), jnp.float32),
                   jax.ShapeDtypeStruct((n_pad, d), jnp.float32),
                   jax.ShapeDtypeStruct((n_pad, 1), jnp.float32)],
    )(p_lo, p_hi, icnt, ocnt, w, b)


def kernel(x, edge_index, graph_len, W1, b1, W2, b2, W3, b3):
    n, d = x.shape
    e = edge_index.shape[1]
    n_pad = -(-(n + 1) // 512) * 512
    e_pad = -(-e // (NT * C * 2)) * (NT * C * 2)

    pad_idx = n + jnp.arange(e_pad - e, dtype=jnp.int32) % (n_pad - n)
    edges = jnp.concatenate(
        [edge_index, jnp.stack([pad_idx, pad_idx])], axis=1)
    edges_r = edges.reshape(2, e_pad // C, C)
    x_pad = jnp.pad(x, ((0, n_pad - n), (0, 0)))

    cnt = _degrees(edges_r, n_pad, e_pad)
    ocnt = cnt[0].reshape(n_pad, 1)
    icnt = cnt[1].reshape(n_pad, 1)

    h_s = _prep(x_pad, ocnt, n, n_pad, d)
    ys, sums = [], []
    for (w, b) in ((W1, b1), (W2, b2), (W3, b3)):
        p_lo, p_hi = _aggregate(h_s, edges_r, n_pad, e_pad, d)
        y, h_s, ysum = _dense(p_lo, p_hi, icnt, ocnt, w,
                              b.reshape(1, d), n, n_pad, d)
        ys.append(y)
        sums.append(ysum)

    x_out = jnp.concatenate([t[:n, 0] for t in sums], axis=0)
    xs_cat = jnp.concatenate([t[:n] for t in ys], axis=1)
    return (x_out, xs_cat)


# speedup vs baseline: 1.1361x; 1.0424x over previous
import functools

import jax
import jax.numpy as jnp
from jax import lax
from jax.experimental import pallas as pl
from jax.experimental.pallas import tpu as pltpu
from jax.experimental.pallas import tpu_sc as plsc

NC = 2
NS = 16
NT = NC * NS
LANE = 16
C = 128


def _mesh():
    return plsc.VectorSubcoreMesh(core_axis_name="c", subcore_axis_name="s")


def _degrees(edges_r, n_pad, e_pad):
    nb = e_pad // C // NS
    nps = n_pad // NS

    @functools.partial(
        pl.kernel,
        out_type=jax.ShapeDtypeStruct((NC, n_pad), jnp.float32),
        mesh=_mesh(),
        scratch_types=[
            pltpu.VMEM((nb, C), jnp.int32),
            pltpu.VMEM((C,), jnp.float32),
            pltpu.VMEM((nps,), jnp.float32),
            pltpu.VMEM_SHARED((n_pad,), jnp.float32),
            pltpu.SemaphoreType.DMA,
        ],
    )
    def deg_kernel(edges_hbm, out_hbm, idx_v, ones_v, zero_v, cnt_sh, sem):
        c = lax.axis_index("c")
        s = lax.axis_index("s")
        cp = pltpu.async_copy(edges_hbm.at[c, pl.ds(s * nb, nb)], idx_v, sem)

        @pl.loop(0, C, step=LANE)
        def _(j):
            ones_v[pl.ds(j, LANE)] = jnp.ones((LANE,), jnp.float32)

        @pl.loop(0, nps, step=LANE)
        def _(j):
            zero_v[pl.ds(j, LANE)] = jnp.zeros((LANE,), jnp.float32)

        pltpu.sync_copy(zero_v, cnt_sh.at[pl.ds(s * nps, nps)])
        cp.wait()
        plsc.subcore_barrier()

        @pl.loop(0, nb)
        def _(j):
            pltpu.sync_copy(ones_v, cnt_sh.at[idx_v.at[j]], add=True)

        plsc.subcore_barrier()
        pltpu.sync_copy(cnt_sh.at[pl.ds(s * nps, nps)],
                        out_hbm.at[c, pl.ds(s * nps, nps)])

    return deg_kernel(edges_r)


def _aggregate(h_s, edges_r, n_pad, e_pad, d):
    edges_r = edges_r.reshape(2, -1)
    cb = 64
    nd = 4
    nch = e_pad // cb // NT
    nq = 4
    nchq = nch // nq
    nps = n_pad // NS

    @functools.partial(
        pl.kernel,
        out_type=[jax.ShapeDtypeStruct((n_pad, d), jnp.float32),
                  jax.ShapeDtypeStruct((n_pad, d), jnp.float32)],
        mesh=_mesh(),
        scratch_types=[
            pltpu.VMEM((nchq, cb), jnp.int32),
            pltpu.VMEM((nchq, cb), jnp.int32),
            [pltpu.VMEM((cb, d), jnp.float32) for _ in range(nd)],
            pltpu.VMEM_SHARED((n_pad, d), jnp.float32),
            [pltpu.SemaphoreType.DMA for _ in range(nd)],
            [pltpu.SemaphoreType.DMA for _ in range(nd)],
        ],
    )
    def agg_kernel(h_hbm, edges_hbm, out_lo, out_hi,
                   sidx, didx, rows, acc_sh, gs, ss):
        c = lax.axis_index("c")
        s = lax.axis_index("s")
        base = (c * NS + s) * nch
        sl = pl.ds(s * nps, nps)

        cp0 = pltpu.async_copy(edges_hbm.at[0, pl.ds(base, nchq)], sidx,
                               gs[0])
        cp1 = pltpu.async_copy(edges_hbm.at[1, pl.ds(base, nchq)], didx,
                               gs[1])

        @pl.loop(0, cb)
        def _(i):
            @pl.loop(0, d, step=LANE)
            def _(j):
                rows[0][i, pl.ds(j, LANE)] = jnp.zeros((LANE,), jnp.float32)

        nzc = nps // cb
        for z in range(nzc):
            pltpu.async_copy(rows[0], acc_sh.at[pl.ds(s * nps + z * cb, cb)],
                             ss[z % nd])
        for z in range(nzc):
            pltpu.make_async_copy(rows[0],
                                  acc_sh.at[pl.ds(s * nps + z * cb, cb)],
                                  ss[z % nd]).wait()
        cp0.wait()
        cp1.wait()

        plsc.subcore_barrier()

        for q in range(nq):
            if q:
                qb = base + q * nchq
                pltpu.sync_copy(edges_hbm.at[0, pl.ds(qb, nchq)], sidx)
                pltpu.sync_copy(edges_hbm.at[1, pl.ds(qb, nchq)], didx)

            for k in range(nd):
                pltpu.async_copy(h_hbm.at[sidx.at[k]], rows[k], gs[k])

            @pl.loop(0, nchq - nd, step=nd)
            def _(j):
                for k in range(nd):
                    pltpu.make_async_copy(h_hbm.at[sidx.at[j + k]],
                                          rows[k], gs[k]).wait()
                    pltpu.async_copy(rows[k], acc_sh.at[didx.at[j + k]],
                                     ss[k], add=True)
                for k in range(nd):
                    pltpu.make_async_copy(rows[k],
                                          acc_sh.at[didx.at[j + k]],
                                          ss[k]).wait()
                    pltpu.async_copy(h_hbm.at[sidx.at[j + nd + k]],
                                     rows[k], gs[k])

            for k in range(nd):
                jj = nchq - nd + k
                pltpu.make_async_copy(h_hbm.at[sidx.at[jj]], rows[k],
                                      gs[k]).wait()
                pltpu.sync_copy(rows[k], acc_sh.at[didx.at[jj]], add=True)

        plsc.subcore_barrier()

        @pl.when(c == 0)
        def _():
            pltpu.sync_copy(acc_sh.at[sl], out_lo.at[sl])

        @pl.when(c != 0)
        def _():
            pltpu.sync_copy(acc_sh.at[sl], out_hi.at[sl])

    return agg_kernel(h_s, edges_r.reshape(2, e_pad // cb, cb))


def _prep(x_pad, ocnt, n, n_pad, d):
    r = 512

    def body(x_ref, oc_ref, hs_ref):
        i = pl.program_id(0)
        rows = i * r + lax.broadcasted_iota(jnp.int32, (r, 1), 0)
        onorm = jnp.where(rows < n,
                          lax.rsqrt(jnp.maximum(oc_ref[...], 1.0)), 0.0)
        hs_ref[...] = x_ref[...] * onorm

    return pl.pallas_call(
        body,
        grid=(n_pad // r,),
        in_specs=[pl.BlockSpec((r, d), lambda i: (i, 0)),
                  pl.BlockSpec((r, 1), lambda i: (i, 0))],
        out_specs=pl.BlockSpec((r, d), lambda i: (i, 0)),
        out_shape=jax.ShapeDtypeStruct((n_pad, d), jnp.float32),
    )(x_pad, ocnt)


def _dense(p_lo, p_hi, icnt, ocnt, w, b, n, n_pad, d):
    r = 1024

    def body(a0_ref, a1_ref, ic_ref, oc_ref, w_ref, b_ref,
             y_ref, ys_ref, sum_ref):
        i = pl.program_id(0)
        rows = i * r + lax.broadcasted_iota(jnp.int32, (r, 1), 0)
        innorm = lax.rsqrt(jnp.maximum(ic_ref[...], 1.0))
        onorm = jnp.where(rows < n,
                          lax.rsqrt(jnp.maximum(oc_ref[...], 1.0)), 0.0)
        a = (a0_ref[...] + a1_ref[...]) * innorm
        yv = jnp.dot(a, w_ref[...], preferred_element_type=jnp.float32)
        yv = jnp.maximum(yv + b_ref[...], 0.0)
        y_ref[...] = yv
        ys_ref[...] = yv * onorm
        sum_ref[...] = jnp.sum(yv, axis=1, keepdims=True)

    return pl.pallas_call(
        body,
        grid=(n_pad // r,),
        in_specs=[pl.BlockSpec((r, d), lambda i: (i, 0)),
                  pl.BlockSpec((r, d), lambda i: (i, 0)),
                  pl.BlockSpec((r, 1), lambda i: (i, 0)),
                  pl.BlockSpec((r, 1), lambda i: (i, 0)),
                  pl.BlockSpec((d, d), lambda i: (0, 0)),
                  pl.BlockSpec((1, d), lambda i: (0, 0))],
        out_specs=[pl.BlockSpec((r, d), lambda i: (i, 0)),
                   pl.BlockSpec((r, d), lambda i: (i, 0)),
                   pl.BlockSpec((r, 1), lambda i: (i, 0))],
        out_shape=[jax.ShapeDtypeStruct((n_pad, d), jnp.float32),
                   jax.ShapeDtypeStruct((n_pad, d), jnp.float32),
                   jax.ShapeDtypeStruct((n_pad, 1), jnp.float32)],
    )(p_lo, p_hi, icnt, ocnt, w, b)


def kernel(x, edge_index, graph_len, W1, b1, W2, b2, W3, b3):
    n, d = x.shape
    e = edge_index.shape[1]
    n_pad = -(-(n + 1) // 512) * 512
    e_pad = -(-e // (NT * C * 2)) * (NT * C * 2)

    pad_idx = n + jnp.arange(e_pad - e, dtype=jnp.int32) % (n_pad - n)
    edges = jnp.concatenate(
        [edge_index, jnp.stack([pad_idx, pad_idx])], axis=1)
    edges_r = edges.reshape(2, e_pad // C, C)
    x_pad = jnp.pad(x, ((0, n_pad - n), (0, 0)))

    cnt = _degrees(edges_r, n_pad, e_pad)
    ocnt = cnt[0].reshape(n_pad, 1)
    icnt = cnt[1].reshape(n_pad, 1)

    h_s = _prep(x_pad, ocnt, n, n_pad, d)
    ys, sums = [], []
    for (w, b) in ((W1, b1), (W2, b2), (W3, b3)):
        p_lo, p_hi = _aggregate(h_s, edges_r, n_pad, e_pad, d)
        y, h_s, ysum = _dense(p_lo, p_hi, icnt, ocnt, w,
                              b.reshape(1, d), n, n_pad, d)
        ys.append(y)
        sums.append(ysum)

    x_out = jnp.concatenate([t[:n, 0] for t in sums], axis=0)
    xs_cat = jnp.concatenate([t[:n] for t in ys], axis=1)
    return (x_out, xs_cat)


# speedup vs baseline: 1.1585x; 1.0197x over previous
import functools

import jax
import jax.numpy as jnp
from jax import lax
from jax.experimental import pallas as pl
from jax.experimental.pallas import tpu as pltpu
from jax.experimental.pallas import tpu_sc as plsc

NC = 2
NS = 16
NT = NC * NS
LANE = 16
C = 128


def _mesh():
    return plsc.VectorSubcoreMesh(core_axis_name="c", subcore_axis_name="s")


def _degrees(edges_r, n_pad, e_pad):
    nb = e_pad // C // NS
    nps = n_pad // NS

    @functools.partial(
        pl.kernel,
        out_type=jax.ShapeDtypeStruct((NC, n_pad), jnp.float32),
        mesh=_mesh(),
        scratch_types=[
            pltpu.VMEM((nb, C), jnp.int32),
            pltpu.VMEM((C,), jnp.float32),
            pltpu.VMEM((nps,), jnp.float32),
            pltpu.VMEM_SHARED((n_pad,), jnp.float32),
            pltpu.SemaphoreType.DMA,
        ],
    )
    def deg_kernel(edges_hbm, out_hbm, idx_v, ones_v, zero_v, cnt_sh, sem):
        c = lax.axis_index("c")
        s = lax.axis_index("s")
        cp = pltpu.async_copy(edges_hbm.at[c, pl.ds(s * nb, nb)], idx_v, sem)

        @pl.loop(0, C, step=LANE)
        def _(j):
            ones_v[pl.ds(j, LANE)] = jnp.ones((LANE,), jnp.float32)

        @pl.loop(0, nps, step=LANE)
        def _(j):
            zero_v[pl.ds(j, LANE)] = jnp.zeros((LANE,), jnp.float32)

        pltpu.sync_copy(zero_v, cnt_sh.at[pl.ds(s * nps, nps)])
        cp.wait()
        plsc.subcore_barrier()

        @pl.loop(0, nb)
        def _(j):
            pltpu.sync_copy(ones_v, cnt_sh.at[idx_v.at[j]], add=True)

        plsc.subcore_barrier()
        pltpu.sync_copy(cnt_sh.at[pl.ds(s * nps, nps)],
                        out_hbm.at[c, pl.ds(s * nps, nps)])

    return deg_kernel(edges_r)


def _aggregate(h_s, edges_r, n_pad, e_pad, d):
    edges_r = edges_r.reshape(2, -1)
    cb = 64
    nd = 4
    nch = e_pad // cb // NT
    nq = 4
    nchq = nch // nq
    nps = n_pad // NS

    @functools.partial(
        pl.kernel,
        out_type=[jax.ShapeDtypeStruct((n_pad, d), jnp.float32),
                  jax.ShapeDtypeStruct((n_pad, d), jnp.float32)],
        mesh=_mesh(),
        scratch_types=[
            pltpu.VMEM((nchq, cb), jnp.int32),
            pltpu.VMEM((nchq, cb), jnp.int32),
            [pltpu.VMEM((cb, d), jnp.float32) for _ in range(nd)],
            pltpu.VMEM_SHARED((n_pad, d), jnp.float32),
            [pltpu.SemaphoreType.DMA for _ in range(nd)],
            [pltpu.SemaphoreType.DMA for _ in range(nd)],
        ],
    )
    def agg_kernel(h_hbm, edges_hbm, out_lo, out_hi,
                   sidx, didx, rows, acc_sh, gs, ss):
        c = lax.axis_index("c")
        s = lax.axis_index("s")
        base = (c * NS + s) * nch
        sl = pl.ds(s * nps, nps)

        cp0 = pltpu.async_copy(edges_hbm.at[0, pl.ds(base, nchq)], sidx,
                               gs[0])
        cp1 = pltpu.async_copy(edges_hbm.at[1, pl.ds(base, nchq)], didx,
                               gs[1])

        @pl.loop(0, cb)
        def _(i):
            @pl.loop(0, d, step=LANE)
            def _(j):
                rows[0][i, pl.ds(j, LANE)] = jnp.zeros((LANE,), jnp.float32)

        nzc = nps // cb
        for z in range(nzc):
            pltpu.async_copy(rows[0], acc_sh.at[pl.ds(s * nps + z * cb, cb)],
                             ss[z % nd])
        for z in range(nzc):
            pltpu.make_async_copy(rows[0],
                                  acc_sh.at[pl.ds(s * nps + z * cb, cb)],
                                  ss[z % nd]).wait()
        cp0.wait()
        cp1.wait()

        plsc.subcore_barrier()

        for q in range(nq):
            if q:
                qb = base + q * nchq
                pltpu.sync_copy(edges_hbm.at[0, pl.ds(qb, nchq)], sidx)
                pltpu.sync_copy(edges_hbm.at[1, pl.ds(qb, nchq)], didx)

            for k in range(nd):
                pltpu.async_copy(h_hbm.at[sidx.at[k]], rows[k], gs[k])

            @pl.loop(0, nchq - nd, step=nd)
            def _(j):
                for k in range(nd):
                    pltpu.make_async_copy(h_hbm.at[sidx.at[j + k]],
                                          rows[k], gs[k]).wait()
                    pltpu.async_copy(rows[k], acc_sh.at[didx.at[j + k]],
                                     ss[k], add=True)
                for k in range(nd):
                    pltpu.make_async_copy(rows[k],
                                          acc_sh.at[didx.at[j + k]],
                                          ss[k]).wait()
                    pltpu.async_copy(h_hbm.at[sidx.at[j + nd + k]],
                                     rows[k], gs[k])

            for k in range(nd):
                jj = nchq - nd + k
                pltpu.make_async_copy(h_hbm.at[sidx.at[jj]], rows[k],
                                      gs[k]).wait()
                pltpu.sync_copy(rows[k], acc_sh.at[didx.at[jj]], add=True)

        plsc.subcore_barrier()

        @pl.when(c == 0)
        def _():
            pltpu.sync_copy(acc_sh.at[sl], out_lo.at[sl])

        @pl.when(c != 0)
        def _():
            pltpu.sync_copy(acc_sh.at[sl], out_hi.at[sl])

    return agg_kernel(h_s, edges_r.reshape(2, e_pad // cb, cb))


def _prep(x_pad, ocnt, n, n_pad, d):
    r = 1024

    def body(x_ref, oc_ref, hs_ref):
        i = pl.program_id(0)
        rows = i * r + lax.broadcasted_iota(jnp.int32, (r, 1), 0)
        onorm = jnp.where(rows < n,
                          lax.rsqrt(jnp.maximum(oc_ref[...], 1.0)), 0.0)
        hs_ref[...] = x_ref[...] * onorm

    return pl.pallas_call(
        body,
        grid=(n_pad // r,),
        in_specs=[pl.BlockSpec((r, d), lambda i: (i, 0)),
                  pl.BlockSpec((r, 1), lambda i: (i, 0))],
        out_specs=pl.BlockSpec((r, d), lambda i: (i, 0)),
        out_shape=jax.ShapeDtypeStruct((n_pad, d), jnp.float32),
    )(x_pad, ocnt)


def _dense(p_lo, p_hi, icnt, ocnt, w, b, n, n_pad, d):
    r = 2048

    def body(a0_ref, a1_ref, ic_ref, oc_ref, w_ref, b_ref,
             y_ref, ys_ref, sum_ref):
        i = pl.program_id(0)
        rows = i * r + lax.broadcasted_iota(jnp.int32, (r, 1), 0)
        innorm = lax.rsqrt(jnp.maximum(ic_ref[...], 1.0))
        onorm = jnp.where(rows < n,
                          lax.rsqrt(jnp.maximum(oc_ref[...], 1.0)), 0.0)
        a = (a0_ref[...] + a1_ref[...]) * innorm
        yv = jnp.dot(a, w_ref[...], preferred_element_type=jnp.float32)
        yv = jnp.maximum(yv + b_ref[...], 0.0)
        y_ref[...] = yv
        ys_ref[...] = yv * onorm
        sum_ref[...] = jnp.sum(yv, axis=1, keepdims=True)

    return pl.pallas_call(
        body,
        grid=(n_pad // r,),
        in_specs=[pl.BlockSpec((r, d), lambda i: (i, 0)),
                  pl.BlockSpec((r, d), lambda i: (i, 0)),
                  pl.BlockSpec((r, 1), lambda i: (i, 0)),
                  pl.BlockSpec((r, 1), lambda i: (i, 0)),
                  pl.BlockSpec((d, d), lambda i: (0, 0)),
                  pl.BlockSpec((1, d), lambda i: (0, 0))],
        out_specs=[pl.BlockSpec((r, d), lambda i: (i, 0)),
                   pl.BlockSpec((r, d), lambda i: (i, 0)),
                   pl.BlockSpec((r, 1), lambda i: (i, 0))],
        out_shape=[jax.ShapeDtypeStruct((n_pad, d), jnp.float32),
                   jax.ShapeDtypeStruct((n_pad, d), jnp.float32),
                   jax.ShapeDtypeStruct((n_pad, 1), jnp.float32)],
    )(p_lo, p_hi, icnt, ocnt, w, b)


def kernel(x, edge_index, graph_len, W1, b1, W2, b2, W3, b3):
    n, d = x.shape
    e = edge_index.shape[1]
    n_pad = -(-(n + 1) // 512) * 512
    e_pad = -(-e // (NT * C * 2)) * (NT * C * 2)

    pad_idx = n + jnp.arange(e_pad - e, dtype=jnp.int32) % (n_pad - n)
    edges = jnp.concatenate(
        [edge_index, jnp.stack([pad_idx, pad_idx])], axis=1)
    edges_r = edges.reshape(2, e_pad // C, C)
    x_pad = jnp.pad(x, ((0, n_pad - n), (0, 0)))

    cnt = _degrees(edges_r, n_pad, e_pad)
    ocnt = cnt[0].reshape(n_pad, 1)
    icnt = cnt[1].reshape(n_pad, 1)

    h_s = _prep(x_pad, ocnt, n, n_pad, d)
    ys, sums = [], []
    for (w, b) in ((W1, b1), (W2, b2), (W3, b3)):
        p_lo, p_hi = _aggregate(h_s, edges_r, n_pad, e_pad, d)
        y, h_s, ysum = _dense(p_lo, p_hi, icnt, ocnt, w,
                              b.reshape(1, d), n, n_pad, d)
        ys.append(y)
        sums.append(ysum)

    x_out = jnp.concatenate([t[:n, 0] for t in sums], axis=0)
    xs_cat = jnp.concatenate([t[:n] for t in ys], axis=1)
    return (x_out, xs_cat)


# speedup vs baseline: 1.1730x; 1.0126x over previous
import functools

import jax
import jax.numpy as jnp
from jax import lax
from jax.experimental import pallas as pl
from jax.experimental.pallas import tpu as pltpu
from jax.experimental.pallas import tpu_sc as plsc

NC = 2
NS = 16
NT = NC * NS
LANE = 16
C = 128


def _mesh():
    return plsc.VectorSubcoreMesh(core_axis_name="c", subcore_axis_name="s")


def _degrees(edges_r, n_pad, e_pad):
    nb = e_pad // C // NS
    nps = n_pad // NS

    @functools.partial(
        pl.kernel,
        out_type=jax.ShapeDtypeStruct((NC, n_pad), jnp.float32),
        mesh=_mesh(),
        scratch_types=[
            pltpu.VMEM((nb, C), jnp.int32),
            pltpu.VMEM((C,), jnp.float32),
            pltpu.VMEM((nps,), jnp.float32),
            pltpu.VMEM_SHARED((n_pad,), jnp.float32),
            pltpu.SemaphoreType.DMA,
        ],
    )
    def deg_kernel(edges_hbm, out_hbm, idx_v, ones_v, zero_v, cnt_sh, sem):
        c = lax.axis_index("c")
        s = lax.axis_index("s")
        cp = pltpu.async_copy(edges_hbm.at[c, pl.ds(s * nb, nb)], idx_v, sem)

        @pl.loop(0, C, step=LANE)
        def _(j):
            ones_v[pl.ds(j, LANE)] = jnp.ones((LANE,), jnp.float32)

        @pl.loop(0, nps, step=LANE)
        def _(j):
            zero_v[pl.ds(j, LANE)] = jnp.zeros((LANE,), jnp.float32)

        pltpu.sync_copy(zero_v, cnt_sh.at[pl.ds(s * nps, nps)])
        cp.wait()
        plsc.subcore_barrier()

        @pl.loop(0, nb)
        def _(j):
            pltpu.sync_copy(ones_v, cnt_sh.at[idx_v.at[j]], add=True)

        plsc.subcore_barrier()
        pltpu.sync_copy(cnt_sh.at[pl.ds(s * nps, nps)],
                        out_hbm.at[c, pl.ds(s * nps, nps)])

    return deg_kernel(edges_r)


def _aggregate(h_s, edges_r, n_pad, e_pad, d):
    edges_r = edges_r.reshape(2, -1)
    cb = 64
    nd = 4
    nch = e_pad // cb // NT
    nq = 4
    nchq = nch // nq
    nps = n_pad // NS

    @functools.partial(
        pl.kernel,
        out_type=[jax.ShapeDtypeStruct((n_pad, d), jnp.float32),
                  jax.ShapeDtypeStruct((n_pad, d), jnp.float32)],
        mesh=_mesh(),
        scratch_types=[
            pltpu.VMEM((nchq, cb), jnp.int32),
            pltpu.VMEM((nchq, cb), jnp.int32),
            [pltpu.VMEM((cb, d), jnp.float32) for _ in range(nd)],
            pltpu.VMEM_SHARED((n_pad, d), jnp.float32),
            [pltpu.SemaphoreType.DMA for _ in range(nd)],
            [pltpu.SemaphoreType.DMA for _ in range(nd)],
        ],
    )
    def agg_kernel(h_hbm, edges_hbm, out_lo, out_hi,
                   sidx, didx, rows, acc_sh, gs, ss):
        c = lax.axis_index("c")
        s = lax.axis_index("s")
        base = (c * NS + s) * nch
        sl = pl.ds(s * nps, nps)

        cp0 = pltpu.async_copy(edges_hbm.at[0, pl.ds(base, nchq)], sidx,
                               gs[0])
        cp1 = pltpu.async_copy(edges_hbm.at[1, pl.ds(base, nchq)], didx,
                               gs[1])

        @pl.loop(0, cb)
        def _(i):
            @pl.loop(0, d, step=LANE)
            def _(j):
                rows[0][i, pl.ds(j, LANE)] = jnp.zeros((LANE,), jnp.float32)

        nzc = nps // cb
        for z in range(nzc):
            pltpu.async_copy(rows[0], acc_sh.at[pl.ds(s * nps + z * cb, cb)],
                             ss[z % nd])
        for z in range(nzc):
            pltpu.make_async_copy(rows[0],
                                  acc_sh.at[pl.ds(s * nps + z * cb, cb)],
                                  ss[z % nd]).wait()
        cp0.wait()
        cp1.wait()

        plsc.subcore_barrier()

        for q in range(nq):
            if q:
                qb = base + q * nchq
                pltpu.sync_copy(edges_hbm.at[0, pl.ds(qb, nchq)], sidx)
                pltpu.sync_copy(edges_hbm.at[1, pl.ds(qb, nchq)], didx)

            for k in range(nd):
                pltpu.async_copy(h_hbm.at[sidx.at[k]], rows[k], gs[k])

            @pl.loop(0, nchq - nd, step=nd)
            def _(j):
                for k in range(nd):
                    pltpu.make_async_copy(h_hbm.at[sidx.at[j + k]],
                                          rows[k], gs[k]).wait()
                    pltpu.async_copy(rows[k], acc_sh.at[didx.at[j + k]],
                                     ss[k], add=True)
                for k in range(nd):
                    pltpu.make_async_copy(rows[k],
                                          acc_sh.at[didx.at[j + k]],
                                          ss[k]).wait()
                    pltpu.async_copy(h_hbm.at[sidx.at[j + nd + k]],
                                     rows[k], gs[k])

            for k in range(nd):
                jj = nchq - nd + k
                pltpu.make_async_copy(h_hbm.at[sidx.at[jj]], rows[k],
                                      gs[k]).wait()
                pltpu.sync_copy(rows[k], acc_sh.at[didx.at[jj]], add=True)

        plsc.subcore_barrier()

        @pl.when(c == 0)
        def _():
            pltpu.sync_copy(acc_sh.at[sl], out_lo.at[sl])

        @pl.when(c != 0)
        def _():
            pltpu.sync_copy(acc_sh.at[sl], out_hi.at[sl])

    return agg_kernel(h_s, edges_r.reshape(2, e_pad // cb, cb))


def _prep(x_pad, ocnt, n, n_pad, d):
    r = 1024

    def body(x_ref, oc_ref, hs_ref):
        i = pl.program_id(0)
        rows = i * r + lax.broadcasted_iota(jnp.int32, (r, 1), 0)
        onorm = jnp.where(rows < n,
                          lax.rsqrt(jnp.maximum(oc_ref[...], 1.0)), 0.0)
        hs_ref[...] = x_ref[...] * onorm

    return pl.pallas_call(
        body,
        grid=(n_pad // r,),
        in_specs=[pl.BlockSpec((r, d), lambda i: (i, 0)),
                  pl.BlockSpec((r, 1), lambda i: (i, 0))],
        out_specs=pl.BlockSpec((r, d), lambda i: (i, 0)),
        out_shape=jax.ShapeDtypeStruct((n_pad, d), jnp.float32),
    )(x_pad, ocnt)


def _dense(p_lo, p_hi, icnt, ocnt, w, b, n, n_pad, d):
    r = 5120

    def body(a0_ref, a1_ref, ic_ref, oc_ref, w_ref, b_ref,
             y_ref, ys_ref, sum_ref):
        i = pl.program_id(0)
        rows = i * r + lax.broadcasted_iota(jnp.int32, (r, 1), 0)
        innorm = lax.rsqrt(jnp.maximum(ic_ref[...], 1.0))
        onorm = jnp.where(rows < n,
                          lax.rsqrt(jnp.maximum(oc_ref[...], 1.0)), 0.0)
        a = (a0_ref[...] + a1_ref[...]) * innorm
        yv = jnp.dot(a, w_ref[...], preferred_element_type=jnp.float32)
        yv = jnp.maximum(yv + b_ref[...], 0.0)
        y_ref[...] = yv
        ys_ref[...] = yv * onorm
        sum_ref[...] = jnp.sum(yv, axis=1, keepdims=True)

    return pl.pallas_call(
        body,
        grid=(n_pad // r,),
        in_specs=[pl.BlockSpec((r, d), lambda i: (i, 0)),
                  pl.BlockSpec((r, d), lambda i: (i, 0)),
                  pl.BlockSpec((r, 1), lambda i: (i, 0)),
                  pl.BlockSpec((r, 1), lambda i: (i, 0)),
                  pl.BlockSpec((d, d), lambda i: (0, 0)),
                  pl.BlockSpec((1, d), lambda i: (0, 0))],
        out_specs=[pl.BlockSpec((r, d), lambda i: (i, 0)),
                   pl.BlockSpec((r, d), lambda i: (i, 0)),
                   pl.BlockSpec((r, 1), lambda i: (i, 0))],
        out_shape=[jax.ShapeDtypeStruct((n_pad, d), jnp.float32),
                   jax.ShapeDtypeStruct((n_pad, d), jnp.float32),
                   jax.ShapeDtypeStruct((n_pad, 1), jnp.float32)],
    )(p_lo, p_hi, icnt, ocnt, w, b)


def kernel(x, edge_index, graph_len, W1, b1, W2, b2, W3, b3):
    n, d = x.shape
    e = edge_index.shape[1]
    n_pad = -(-(n + 1) // 512) * 512
    e_pad = -(-e // (NT * C * 2)) * (NT * C * 2)

    pad_idx = n + jnp.arange(e_pad - e, dtype=jnp.int32) % (n_pad - n)
    edges = jnp.concatenate(
        [edge_index, jnp.stack([pad_idx, pad_idx])], axis=1)
    edges_r = edges.reshape(2, e_pad // C, C)
    x_pad = jnp.pad(x, ((0, n_pad - n), (0, 0)))

    cnt = _degrees(edges_r, n_pad, e_pad)
    ocnt = cnt[0].reshape(n_pad, 1)
    icnt = cnt[1].reshape(n_pad, 1)

    h_s = _prep(x_pad, ocnt, n, n_pad, d)
    ys, sums = [], []
    for (w, b) in ((W1, b1), (W2, b2), (W3, b3)):
        p_lo, p_hi = _aggregate(h_s, edges_r, n_pad, e_pad, d)
        y, h_s, ysum = _dense(p_lo, p_hi, icnt, ocnt, w,
                              b.reshape(1, d), n, n_pad, d)
        ys.append(y)
        sums.append(ysum)

    x_out = jnp.concatenate([t[:n, 0] for t in sums], axis=0)
    xs_cat = jnp.concatenate([t[:n] for t in ys], axis=1)
    return (x_out, xs_cat)
